# Initial kernel scaffold; baseline (speedup 1.0000x reference)
import jax
import jax.numpy as jnp
from jax.experimental import pallas as pl


def kernel(input, target):
    raise NotImplementedError("write your pallas kernel here")


# speedup vs baseline: 39.0733x; 39.0733x over previous
import jax
import jax.numpy as jnp
from jax import lax
from jax.experimental import pallas as pl
from jax.experimental.pallas import tpu as pltpu
from jax.experimental.pallas import tpu_sc as plsc

_B = 16384
_N = 1000
_TOPK = 7

_SC_ROWS = 8192
_TC_ROWS = _B - _SC_ROWS

_NW = 32
_RPW = _SC_ROWS // _NW
_DMAB = 128
_NDMA = _RPW // _DMAB
_NSUB = _DMAB // 16
_UNROLL = 8

_TC_BLOCK = 1024
_TC_GRID = _TC_ROWS // _TC_BLOCK
_TC_BLOCK0 = _SC_ROWS // _TC_BLOCK


_SORT8 = (
    (0, 1), (2, 3), (4, 5), (6, 7), (0, 2), (1, 3), (4, 6), (5, 7),
    (1, 2), (5, 6), (0, 4), (1, 5), (2, 6), (3, 7), (2, 4), (3, 5),
    (1, 2), (3, 4), (5, 6),
)
_BITONIC8 = (
    (0, 4), (1, 5), (2, 6), (3, 7), (0, 2), (1, 3), (4, 6), (5, 7),
    (0, 1), (2, 3), (4, 5), (6, 7),
)


def _net_desc(a, net):
    for i, j in net:
        hi = jnp.maximum(a[i], a[j])
        a[j] = jnp.minimum(a[i], a[j])
        a[i] = hi
    return a


def _sc_body(x_hbm, tgt_hbm, out_hbm, buf, tgt_v, out_v):
    wid = lax.axis_index("s") * 2 + lax.axis_index("c")
    base = wid * _RPW
    pltpu.sync_copy(tgt_hbm.at[pl.ds(base, _RPW)], tgt_v)

    lane = lax.iota(jnp.int32, 16)
    zero16 = jnp.zeros((16,), jnp.float32)
    ninf = jnp.full((16,), -jnp.inf, jnp.float32)

    def slab_body(tb, acc):
        pltpu.sync_copy(
            x_hbm.at[pl.ds(0, _N), pl.ds(base + tb * _DMAB, _DMAB)], buf
        )
        for st in range(_NSUB):
            tgt16 = tgt_v[pl.ds(pl.multiple_of(tb * _DMAB + st * 16, 16), 16)]
            tv = plsc.load_gather(buf, [tgt16, lane + st * 16])

            def feat_body(fb, carry, _st=st):
                s_acc, t1, t2, t3, t4, t5, t6, t7 = carry
                top = [t1, t2, t3, t4, t5, t6, t7]
                xs = []
                for u in range(_UNROLL):
                    x = buf[fb * _UNROLL + u, pl.ds(_st * 16, 16)]
                    s_acc = s_acc + jnp.maximum(x - tv, 0.0)
                    xs.append(x)
                xs = _net_desc(xs, _SORT8)
                h = [jnp.maximum(top[i], xs[7 - i]) for i in range(_TOPK)]
                h.append(xs[0])
                h = _net_desc(h, _BITONIC8)
                return (s_acc, *h[:_TOPK])

            carry = lax.fori_loop(
                0, _N // _UNROLL, feat_body, (zero16,) + (ninf,) * _TOPK
            )
            s_acc = carry[0]
            for t in carry[1:]:
                s_acc = s_acc - jnp.maximum(t - tv, 0.0)
            acc = acc + s_acc
        return acc

    acc = lax.fori_loop(0, _NDMA, slab_body, zero16)
    out_v[...] = acc
    pltpu.sync_copy(out_v, out_hbm.at[wid])


def _sc_partial(xt, tgt):
    mesh = plsc.VectorSubcoreMesh(core_axis_name="c", subcore_axis_name="s")
    partials = pl.kernel(
        _sc_body,
        out_type=jax.ShapeDtypeStruct((_NW, 16), jnp.float32),
        mesh=mesh,
        compiler_params=pltpu.CompilerParams(needs_layout_passes=False),
        scratch_types=[
            pltpu.VMEM((_N, _DMAB), jnp.float32),
            pltpu.VMEM((_RPW,), jnp.int32),
            pltpu.VMEM((16,), jnp.float32),
        ],
    )(xt, tgt)
    return jnp.sum(partials)


def _tc_block_body(x_ref, t_ref, out_ref):
    v = x_ref[...]
    tgt = t_ref[...]

    feat = jax.lax.broadcasted_iota(jnp.int32, v.shape, 0)
    is_tgt = feat == tgt
    tsc = jnp.max(jnp.where(is_tgt, v, v - 100000.0), axis=0, keepdims=True)

    s_all = jnp.sum(jnp.maximum(v, tsc), axis=0) - float(_N) * tsc[0, :]

    neg_inf = jnp.float32(-jnp.inf)
    remaining = jnp.full((v.shape[1],), float(_TOPK), dtype=jnp.float32)
    top_sum = jnp.zeros((v.shape[1],), dtype=jnp.float32)
    for k in range(_TOPK):
        if k == 0:
            theta = jnp.max(v, axis=0, keepdims=True)
        else:
            masked = jnp.where(v < theta, v, neg_inf)
            theta = jnp.max(masked, axis=0, keepdims=True)
        cnt = jnp.sum(jnp.where(v == theta, 1.0, 0.0), axis=0)
        take = jnp.minimum(cnt, remaining)
        top_sum += take * jnp.maximum(theta[0, :] - tsc[0, :], 0.0)
        remaining -= take

    block_loss = jnp.sum(s_all - top_sum).reshape(1, 1)

    @pl.when(pl.program_id(0) == 0)
    def _init():
        out_ref[...] = jnp.zeros((1, 1), dtype=jnp.float32)

    out_ref[...] += block_loss


def _tc_partial(xt, tgt_row):
    out = pl.pallas_call(
        _tc_block_body,
        grid=(_TC_GRID,),
        in_specs=[
            pl.BlockSpec((_N, _TC_BLOCK), lambda i: (0, i + _TC_BLOCK0)),
            pl.BlockSpec((1, _TC_BLOCK), lambda i: (0, i + _TC_BLOCK0)),
        ],
        out_specs=pl.BlockSpec((1, 1), lambda i: (0, 0)),
        out_shape=jax.ShapeDtypeStruct((1, 1), jnp.float32),
    )(xt, tgt_row)
    return out[0, 0]


@jax.jit
def _cluster_margin_loss(x, tgt):
    xt = x.T
    tgt_row = tgt.reshape(1, _B)
    sc = _sc_partial(xt, tgt.reshape(_B))
    tc = _tc_partial(xt, tgt_row)
    return (sc + tc) / (_B * 500.0)


def kernel(input, target):
    return _cluster_margin_loss(input, target)
